# baseline (speedup 1.0000x reference)
import functools
import math

import jax
import jax.numpy as jnp
from jax.experimental import pallas as pl
import jax.experimental.pallas.tpu as pltpu

N_HEADS = 16
N_KV_HEADS = 4
HEAD_SIZE = 128
MAX_SEQLEN = 2048
SPARSE_BLOCK = 64
VLLM_BLOCK = 16
LOCAL_BLOCKS = 8
VERT_STRIDE = 4
NUM_SEQS = 32
BLOCKS_PER_SEQ = MAX_SEQLEN // VLLM_BLOCK
PAGES_PER_SPARSE = SPARSE_BLOCK // VLLM_BLOCK
NUM_SPARSE_BLOCKS = MAX_SEQLEN // SPARSE_BLOCK
MAX_ACTIVE = LOCAL_BLOCKS + (NUM_SPARSE_BLOCKS - LOCAL_BLOCKS) // VERT_STRIDE

R = N_HEADS // N_KV_HEADS
SM_SCALE = 1.0 / math.sqrt(HEAD_SIZE)
NEG_INF = -1e30


def _active_block(s, j, ctx_ref):
    q_blk = (ctx_ref[s] - 1) // SPARSE_BLOCK
    n_strided = jnp.maximum((q_blk - (LOCAL_BLOCKS - 1)) // VERT_STRIDE, 0)
    local_start = jnp.maximum(q_blk - (LOCAL_BLOCKS - 1), 0)
    b = jnp.where(j < n_strided,
                  VERT_STRIDE * j + (VERT_STRIDE - 1),
                  local_start + (j - n_strided))
    return jnp.minimum(b, q_blk), q_blk, n_strided


def _kv_index_map(s, j, ctx_ref, chunk_ref):
    b, _, _ = _active_block(s, j, ctx_ref)
    return (chunk_ref[s, b], 0, 0, 0)


def _q_index_map(s, j, ctx_ref, chunk_ref):
    return (s, 0, 0)


def _attn_kernel(ctx_ref, chunk_ref, q_ref, k_ref, v_ref, out_ref,
                 m_ref, l_ref, acc_ref):
    s = pl.program_id(0)
    j = pl.program_id(1)

    b, q_blk, n_strided = _active_block(s, j, ctx_ref)
    n_local = jnp.minimum(q_blk + 1, LOCAL_BLOCKS)
    n_active = n_strided + n_local

    @pl.when(j == 0)
    def _init():
        m_ref[...] = jnp.full_like(m_ref, NEG_INF)
        l_ref[...] = jnp.zeros_like(l_ref)
        acc_ref[...] = jnp.zeros_like(acc_ref)

    @pl.when(j < n_active)
    def _step():
        q3 = q_ref[0].reshape(N_KV_HEADS, R, HEAD_SIZE)
        kb = k_ref[...]
        vb = v_ref[...]
        scores = jax.lax.dot_general(
            q3, kb,
            dimension_numbers=(((2,), (2,)), ((0,), (1,))),
            preferred_element_type=jnp.float32) * SM_SCALE

        ctx = ctx_ref[s]
        t_global = (b * SPARSE_BLOCK
                    + jax.lax.broadcasted_iota(jnp.int32, scores.shape, 2)
                    * VLLM_BLOCK
                    + jax.lax.broadcasted_iota(jnp.int32, scores.shape, 3))
        scores = jnp.where(t_global < ctx, scores, NEG_INF)

        m_prev = m_ref[...]
        l_prev = l_ref[...]
        m_cur = jnp.max(scores, axis=(2, 3))
        m_new = jnp.maximum(m_prev, m_cur)
        alpha = jnp.exp(m_prev - m_new)
        p = jnp.exp(scores - m_new[:, :, None, None])
        l_new = l_prev * alpha + jnp.sum(p, axis=(2, 3))
        pv = jax.lax.dot_general(
            p[:, :, 0, :], vb[0],
            dimension_numbers=(((2,), (2,)), ((0,), (0,))),
            preferred_element_type=jnp.float32)
        for c in range(1, PAGES_PER_SPARSE):
            pv = pv + jax.lax.dot_general(
                p[:, :, c, :], vb[c],
                dimension_numbers=(((2,), (2,)), ((0,), (0,))),
                preferred_element_type=jnp.float32)
        acc_ref[...] = acc_ref[...] * alpha[..., None] + pv
        m_ref[...] = m_new
        l_ref[...] = l_new

    @pl.when(j == MAX_ACTIVE - 1)
    def _finalize():
        out = acc_ref[...] / l_ref[...][..., None]
        out_ref[0] = out.reshape(N_HEADS, HEAD_SIZE)


@jax.jit
def kernel(q, k, v, block_tables, context_lens):
    chunk_tables = block_tables[:, ::PAGES_PER_SPARSE] // PAGES_PER_SPARSE

    grid_spec = pltpu.PrefetchScalarGridSpec(
        num_scalar_prefetch=2,
        grid=(NUM_SEQS, MAX_ACTIVE),
        in_specs=[
            pl.BlockSpec((1, N_HEADS, HEAD_SIZE), _q_index_map),
            pl.BlockSpec((PAGES_PER_SPARSE, N_KV_HEADS, HEAD_SIZE,
                          VLLM_BLOCK), _kv_index_map),
            pl.BlockSpec((PAGES_PER_SPARSE, N_KV_HEADS, HEAD_SIZE,
                          VLLM_BLOCK), _kv_index_map),
        ],
        out_specs=pl.BlockSpec((1, N_HEADS, HEAD_SIZE),
                               lambda s, j, ctx, chunk: (s, 0, 0)),
        scratch_shapes=[
            pltpu.VMEM((N_KV_HEADS, R), jnp.float32),
            pltpu.VMEM((N_KV_HEADS, R), jnp.float32),
            pltpu.VMEM((N_KV_HEADS, R, HEAD_SIZE), jnp.float32),
        ],
    )

    return pl.pallas_call(
        _attn_kernel,
        grid_spec=grid_spec,
        out_shape=jax.ShapeDtypeStruct((NUM_SEQS, N_HEADS, HEAD_SIZE),
                                       jnp.float32),
    )(context_lens, chunk_tables, q, k, v)


# speedup vs baseline: 1.0566x; 1.0566x over previous
import functools
import math

import jax
import jax.numpy as jnp
from jax.experimental import pallas as pl
import jax.experimental.pallas.tpu as pltpu

N_HEADS = 16
N_KV_HEADS = 4
HEAD_SIZE = 128
MAX_SEQLEN = 2048
SPARSE_BLOCK = 64
VLLM_BLOCK = 16
LOCAL_BLOCKS = 8
VERT_STRIDE = 4
NUM_SEQS = 32
BLOCKS_PER_SEQ = MAX_SEQLEN // VLLM_BLOCK
PAGES_PER_SPARSE = SPARSE_BLOCK // VLLM_BLOCK
NUM_SPARSE_BLOCKS = MAX_SEQLEN // SPARSE_BLOCK
MAX_ACTIVE = LOCAL_BLOCKS + (NUM_SPARSE_BLOCKS - LOCAL_BLOCKS) // VERT_STRIDE

R = N_HEADS // N_KV_HEADS
SM_SCALE = 1.0 / math.sqrt(HEAD_SIZE)
NEG_INF = -1e30


def _kv_index_map(s, j, cidx_ref, blk_ref, nact_ref, ctx_ref):
    return (cidx_ref[s, j], 0, 0, 0)


def _q_index_map(s, j, cidx_ref, blk_ref, nact_ref, ctx_ref):
    return (s, 0, 0)


def _attn_kernel(cidx_ref, blk_ref, nact_ref, ctx_ref,
                 qbd_ref, k_ref, v_ref, out_ref,
                 m_ref, l_ref, acc_ref):
    s = pl.program_id(0)
    j = pl.program_id(1)

    @pl.when(j == 0)
    def _init():
        m_ref[...] = jnp.full_like(m_ref, NEG_INF)
        l_ref[...] = jnp.zeros_like(l_ref)
        acc_ref[...] = jnp.zeros_like(acc_ref)

    @pl.when(j < nact_ref[s])
    def _step():
        qbd = qbd_ref[0]
        kb = k_ref[...].reshape(PAGES_PER_SPARSE, N_KV_HEADS * HEAD_SIZE,
                                VLLM_BLOCK)
        vb = v_ref[...]
        ctx = ctx_ref[s]
        t_base = blk_ref[s, j] * SPARSE_BLOCK

        scores = []
        for c in range(PAGES_PER_SPARSE):
            sc = jax.lax.dot_general(
                qbd, kb[c],
                dimension_numbers=(((1,), (0,)), ((), ())),
                preferred_element_type=jnp.float32)
            t_global = (t_base + c * VLLM_BLOCK
                        + jax.lax.broadcasted_iota(jnp.int32, sc.shape, 1))
            scores.append(jnp.where(t_global < ctx, sc, NEG_INF))

        m_prev = m_ref[...]
        m_cur = jnp.max(scores[0], axis=1, keepdims=True)
        for c in range(1, PAGES_PER_SPARSE):
            m_cur = jnp.maximum(m_cur, jnp.max(scores[c], axis=1,
                                               keepdims=True))
        m_new = jnp.maximum(m_prev, m_cur)
        alpha = jnp.exp(m_prev - m_new)

        l_new = l_ref[...] * alpha
        pv = None
        for c in range(PAGES_PER_SPARSE):
            p = jnp.exp(scores[c] - m_new)
            l_new = l_new + jnp.sum(p, axis=1, keepdims=True)
            d = jax.lax.dot_general(
                p.reshape(N_KV_HEADS, R, VLLM_BLOCK), vb[c],
                dimension_numbers=(((2,), (2,)), ((0,), (0,))),
                preferred_element_type=jnp.float32)
            pv = d if pv is None else pv + d

        acc_ref[...] = (acc_ref[...] * alpha.reshape(N_KV_HEADS, R, 1) + pv)
        m_ref[...] = m_new
        l_ref[...] = l_new

    @pl.when(j == MAX_ACTIVE - 1)
    def _finalize():
        out = acc_ref[...] / l_ref[...].reshape(N_KV_HEADS, R, 1)
        out_ref[0] = out.reshape(N_HEADS, HEAD_SIZE)


@jax.jit
def kernel(q, k, v, block_tables, context_lens):
    qb = (context_lens - 1) // SPARSE_BLOCK
    n_str = jnp.maximum((qb - (LOCAL_BLOCKS - 1)) // VERT_STRIDE, 0)
    local_start = jnp.maximum(qb - (LOCAL_BLOCKS - 1), 0)
    n_act = n_str + jnp.minimum(qb + 1, LOCAL_BLOCKS)
    jj = jnp.arange(MAX_ACTIVE, dtype=jnp.int32)[None, :]
    blk = jnp.where(jj < n_str[:, None],
                    VERT_STRIDE * jj + (VERT_STRIDE - 1),
                    local_start[:, None] + (jj - n_str[:, None]))
    blk = jnp.minimum(blk, qb[:, None]).astype(jnp.int32)
    cidx = (jnp.take_along_axis(block_tables, blk * PAGES_PER_SPARSE,
                                axis=1) // PAGES_PER_SPARSE).astype(jnp.int32)

    eye = jnp.eye(N_KV_HEADS, dtype=q.dtype)
    qbd = (q.reshape(NUM_SEQS, N_KV_HEADS, R, 1, HEAD_SIZE)
           * eye[None, :, None, :, None] * SM_SCALE)
    qbd = qbd.reshape(NUM_SEQS, N_HEADS, N_KV_HEADS * HEAD_SIZE)

    grid_spec = pltpu.PrefetchScalarGridSpec(
        num_scalar_prefetch=4,
        grid=(NUM_SEQS, MAX_ACTIVE),
        in_specs=[
            pl.BlockSpec((1, N_HEADS, N_KV_HEADS * HEAD_SIZE), _q_index_map),
            pl.BlockSpec((PAGES_PER_SPARSE, N_KV_HEADS, HEAD_SIZE,
                          VLLM_BLOCK), _kv_index_map),
            pl.BlockSpec((PAGES_PER_SPARSE, N_KV_HEADS, HEAD_SIZE,
                          VLLM_BLOCK), _kv_index_map),
        ],
        out_specs=pl.BlockSpec((1, N_HEADS, HEAD_SIZE),
                               lambda s, j, *_: (s, 0, 0)),
        scratch_shapes=[
            pltpu.VMEM((N_HEADS, 1), jnp.float32),
            pltpu.VMEM((N_HEADS, 1), jnp.float32),
            pltpu.VMEM((N_KV_HEADS, R, HEAD_SIZE), jnp.float32),
        ],
    )

    return pl.pallas_call(
        _attn_kernel,
        grid_spec=grid_spec,
        out_shape=jax.ShapeDtypeStruct((NUM_SEQS, N_HEADS, HEAD_SIZE),
                                       jnp.float32),
    )(cidx, blk, n_act, context_lens, qbd, k, v)


# speedup vs baseline: 1.1603x; 1.0982x over previous
import functools
import math

import jax
import jax.numpy as jnp
from jax.experimental import pallas as pl
import jax.experimental.pallas.tpu as pltpu

N_HEADS = 16
N_KV_HEADS = 4
HEAD_SIZE = 128
MAX_SEQLEN = 2048
SPARSE_BLOCK = 64
VLLM_BLOCK = 16
LOCAL_BLOCKS = 8
VERT_STRIDE = 4
NUM_SEQS = 32
BLOCKS_PER_SEQ = MAX_SEQLEN // VLLM_BLOCK
PAGES_PER_SPARSE = SPARSE_BLOCK // VLLM_BLOCK
NUM_SPARSE_BLOCKS = MAX_SEQLEN // SPARSE_BLOCK
MAX_ACTIVE = LOCAL_BLOCKS + (NUM_SPARSE_BLOCKS - LOCAL_BLOCKS) // VERT_STRIDE

R = N_HEADS // N_KV_HEADS
SM_SCALE = 1.0 / math.sqrt(HEAD_SIZE)
NEG_INF = -1e30


def _attn_kernel(cidx_ref, blk_ref, nact_ref, ctx_ref,
                 qbd_ref, k_hbm, v_hbm, out_ref,
                 kbuf, vbuf, m_ref, l_ref, acc_ref, ksem, vsem):
    s = pl.program_id(0)
    n = nact_ref[s]
    ctx = ctx_ref[s]

    for j in range(MAX_ACTIVE):
        @pl.when(j < n)
        def _start(j=j):
            base = cidx_ref[s, j] * PAGES_PER_SPARSE
            pltpu.make_async_copy(
                k_hbm.at[pl.ds(base, PAGES_PER_SPARSE)], kbuf.at[j],
                ksem.at[j]).start()
            pltpu.make_async_copy(
                v_hbm.at[pl.ds(base, PAGES_PER_SPARSE)], vbuf.at[j],
                vsem.at[j]).start()

    m_ref[...] = jnp.full_like(m_ref, NEG_INF)
    l_ref[...] = jnp.zeros_like(l_ref)
    acc_ref[...] = jnp.zeros_like(acc_ref)

    qbd = qbd_ref[0]

    for j in range(MAX_ACTIVE):
        @pl.when(j < n)
        def _step(j=j):
            pltpu.make_async_copy(
                k_hbm.at[pl.ds(0, PAGES_PER_SPARSE)], kbuf.at[j],
                ksem.at[j]).wait()
            pltpu.make_async_copy(
                v_hbm.at[pl.ds(0, PAGES_PER_SPARSE)], vbuf.at[j],
                vsem.at[j]).wait()
            kb = kbuf[j].reshape(PAGES_PER_SPARSE, N_KV_HEADS * HEAD_SIZE,
                                 VLLM_BLOCK)
            vb = vbuf[j]
            t_base = blk_ref[s, j] * SPARSE_BLOCK

            scores = []
            for c in range(PAGES_PER_SPARSE):
                sc = jax.lax.dot_general(
                    qbd, kb[c],
                    dimension_numbers=(((1,), (0,)), ((), ())),
                    preferred_element_type=jnp.float32)
                t_global = (t_base + c * VLLM_BLOCK
                            + jax.lax.broadcasted_iota(jnp.int32, sc.shape, 1))
                scores.append(jnp.where(t_global < ctx, sc, NEG_INF))

            m_prev = m_ref[...]
            m_cur = jnp.max(scores[0], axis=1, keepdims=True)
            for c in range(1, PAGES_PER_SPARSE):
                m_cur = jnp.maximum(m_cur, jnp.max(scores[c], axis=1,
                                                   keepdims=True))
            m_new = jnp.maximum(m_prev, m_cur)
            alpha = jnp.exp(m_prev - m_new)

            l_new = l_ref[...] * alpha
            pv = None
            for c in range(PAGES_PER_SPARSE):
                p = jnp.exp(scores[c] - m_new)
                l_new = l_new + jnp.sum(p, axis=1, keepdims=True)
                d = jax.lax.dot_general(
                    p.reshape(N_KV_HEADS, R, VLLM_BLOCK), vb[c],
                    dimension_numbers=(((2,), (2,)), ((0,), (0,))),
                    preferred_element_type=jnp.float32)
                pv = d if pv is None else pv + d

            acc_ref[...] = (acc_ref[...] * alpha.reshape(N_KV_HEADS, R, 1)
                            + pv)
            m_ref[...] = m_new
            l_ref[...] = l_new

    out = acc_ref[...] / l_ref[...].reshape(N_KV_HEADS, R, 1)
    out_ref[0] = out.reshape(N_HEADS, HEAD_SIZE)


@jax.jit
def kernel(q, k, v, block_tables, context_lens):
    qb = (context_lens - 1) // SPARSE_BLOCK
    n_str = jnp.maximum((qb - (LOCAL_BLOCKS - 1)) // VERT_STRIDE, 0)
    local_start = jnp.maximum(qb - (LOCAL_BLOCKS - 1), 0)
    n_act = (n_str + jnp.minimum(qb + 1, LOCAL_BLOCKS)).astype(jnp.int32)
    jj = jnp.arange(MAX_ACTIVE, dtype=jnp.int32)[None, :]
    blk = jnp.where(jj < n_str[:, None],
                    VERT_STRIDE * jj + (VERT_STRIDE - 1),
                    local_start[:, None] + (jj - n_str[:, None]))
    blk = jnp.minimum(blk, qb[:, None]).astype(jnp.int32)
    cidx = (jnp.take_along_axis(block_tables, blk * PAGES_PER_SPARSE,
                                axis=1) // PAGES_PER_SPARSE).astype(jnp.int32)

    eye = jnp.eye(N_KV_HEADS, dtype=q.dtype)
    qbd = (q.reshape(NUM_SEQS, N_KV_HEADS, R, 1, HEAD_SIZE)
           * eye[None, :, None, :, None] * SM_SCALE)
    qbd = qbd.reshape(NUM_SEQS, N_HEADS, N_KV_HEADS * HEAD_SIZE)

    grid_spec = pltpu.PrefetchScalarGridSpec(
        num_scalar_prefetch=4,
        grid=(NUM_SEQS,),
        in_specs=[
            pl.BlockSpec((1, N_HEADS, N_KV_HEADS * HEAD_SIZE),
                         lambda s, *_: (s, 0, 0)),
            pl.BlockSpec(memory_space=pl.ANY),
            pl.BlockSpec(memory_space=pl.ANY),
        ],
        out_specs=pl.BlockSpec((1, N_HEADS, HEAD_SIZE),
                               lambda s, *_: (s, 0, 0)),
        scratch_shapes=[
            pltpu.VMEM((MAX_ACTIVE, PAGES_PER_SPARSE, N_KV_HEADS, HEAD_SIZE,
                        VLLM_BLOCK), jnp.float32),
            pltpu.VMEM((MAX_ACTIVE, PAGES_PER_SPARSE, N_KV_HEADS, HEAD_SIZE,
                        VLLM_BLOCK), jnp.float32),
            pltpu.VMEM((N_HEADS, 1), jnp.float32),
            pltpu.VMEM((N_HEADS, 1), jnp.float32),
            pltpu.VMEM((N_KV_HEADS, R, HEAD_SIZE), jnp.float32),
            pltpu.SemaphoreType.DMA((MAX_ACTIVE,)),
            pltpu.SemaphoreType.DMA((MAX_ACTIVE,)),
        ],
    )

    return pl.pallas_call(
        _attn_kernel,
        grid_spec=grid_spec,
        out_shape=jax.ShapeDtypeStruct((NUM_SEQS, N_HEADS, HEAD_SIZE),
                                       jnp.float32),
    )(cidx, blk, n_act, context_lens, qbd, k, v)


# speedup vs baseline: 4.8362x; 4.1679x over previous
import math

import jax
import jax.numpy as jnp
from jax.experimental import pallas as pl
import jax.experimental.pallas.tpu as pltpu

N_HEADS = 16
N_KV_HEADS = 4
HEAD_SIZE = 128
MAX_SEQLEN = 2048
SPARSE_BLOCK = 64
VLLM_BLOCK = 16
LOCAL_BLOCKS = 8
VERT_STRIDE = 4
NUM_SEQS = 32
PAGES_PER_SPARSE = SPARSE_BLOCK // VLLM_BLOCK
NUM_SPARSE_BLOCKS = MAX_SEQLEN // SPARSE_BLOCK
MAX_ACTIVE = LOCAL_BLOCKS + (NUM_SPARSE_BLOCKS - LOCAL_BLOCKS) // VERT_STRIDE
PAGES_PER_SEQ = MAX_ACTIVE * PAGES_PER_SPARSE
T_ACT = MAX_ACTIVE * SPARSE_BLOCK

R = N_HEADS // N_KV_HEADS
SM_SCALE = 1.0 / math.sqrt(HEAD_SIZE)
NEG_INF = -1e30


def _attn_kernel(ctx_ref, q_ref, tok_ref, k_hbm, v_hbm, out_ref,
                 kbuf, vbuf, ksem, vsem):
    s = pl.program_id(0)
    base = s * PAGES_PER_SEQ
    pltpu.make_async_copy(k_hbm.at[pl.ds(base, PAGES_PER_SEQ)], kbuf,
                          ksem).start()
    pltpu.make_async_copy(v_hbm.at[pl.ds(base, PAGES_PER_SEQ)], vbuf,
                          vsem).start()
    pltpu.make_async_copy(k_hbm.at[pl.ds(0, PAGES_PER_SEQ)], kbuf,
                          ksem).wait()
    pltpu.make_async_copy(v_hbm.at[pl.ds(0, PAGES_PER_SEQ)], vbuf,
                          vsem).wait()

    valid = tok_ref[0, 0] < ctx_ref[s]
    outs = []
    for h in range(N_KV_HEADS):
        kh = kbuf[:, h].reshape(T_ACT, HEAD_SIZE)
        vh = vbuf[:, h].reshape(T_ACT, HEAD_SIZE)
        qh = q_ref[0, h]
        sc = jax.lax.dot_general(
            qh, kh,
            dimension_numbers=(((1,), (1,)), ((), ())),
            preferred_element_type=jnp.float32)
        sc = jnp.where(valid[None, :], sc, NEG_INF)
        m = jnp.max(sc, axis=1, keepdims=True)
        p = jnp.exp(sc - m)
        l = jnp.sum(p, axis=1, keepdims=True)
        pv = jax.lax.dot_general(
            p, vh,
            dimension_numbers=(((1,), (0,)), ((), ())),
            preferred_element_type=jnp.float32)
        outs.append(pv / l)
    out_ref[0] = jnp.concatenate(outs, axis=0)


@jax.jit
def kernel(q, k, v, block_tables, context_lens):
    qb = (context_lens - 1) // SPARSE_BLOCK
    n_str = jnp.maximum((qb - (LOCAL_BLOCKS - 1)) // VERT_STRIDE, 0)
    local_start = jnp.maximum(qb - (LOCAL_BLOCKS - 1), 0)
    n_act = n_str + jnp.minimum(qb + 1, LOCAL_BLOCKS)
    jj = jnp.arange(MAX_ACTIVE, dtype=jnp.int32)[None, :]
    blk = jnp.where(jj < n_str[:, None],
                    VERT_STRIDE * jj + (VERT_STRIDE - 1),
                    local_start[:, None] + (jj - n_str[:, None]))
    blk = jnp.minimum(blk, qb[:, None]).astype(jnp.int32)

    tok = (blk[:, :, None] * SPARSE_BLOCK
           + jnp.arange(SPARSE_BLOCK, dtype=jnp.int32))
    tok = jnp.where((jj < n_act[:, None])[:, :, None], tok, jnp.int32(1 << 30))
    tok = tok.reshape(NUM_SEQS, 1, T_ACT)

    pages = (jnp.take_along_axis(block_tables, blk * PAGES_PER_SPARSE,
                                 axis=1)[..., None]
             + jnp.arange(PAGES_PER_SPARSE, dtype=jnp.int32))
    pages = pages.reshape(-1)
    kact = jnp.take(k, pages, axis=0).transpose(0, 1, 3, 2)
    vact = jnp.take(v, pages, axis=0).transpose(0, 1, 3, 2)

    q3 = (q * SM_SCALE).reshape(NUM_SEQS, N_KV_HEADS, R, HEAD_SIZE)

    grid_spec = pltpu.PrefetchScalarGridSpec(
        num_scalar_prefetch=1,
        grid=(NUM_SEQS,),
        in_specs=[
            pl.BlockSpec((1, N_KV_HEADS, R, HEAD_SIZE),
                         lambda s, *_: (s, 0, 0, 0)),
            pl.BlockSpec((1, 1, T_ACT), lambda s, *_: (s, 0, 0)),
            pl.BlockSpec(memory_space=pl.ANY),
            pl.BlockSpec(memory_space=pl.ANY),
        ],
        out_specs=pl.BlockSpec((1, N_HEADS, HEAD_SIZE),
                               lambda s, *_: (s, 0, 0)),
        scratch_shapes=[
            pltpu.VMEM((PAGES_PER_SEQ, N_KV_HEADS, VLLM_BLOCK, HEAD_SIZE),
                       jnp.float32),
            pltpu.VMEM((PAGES_PER_SEQ, N_KV_HEADS, VLLM_BLOCK, HEAD_SIZE),
                       jnp.float32),
            pltpu.SemaphoreType.DMA,
            pltpu.SemaphoreType.DMA,
        ],
    )

    return pl.pallas_call(
        _attn_kernel,
        grid_spec=grid_spec,
        out_shape=jax.ShapeDtypeStruct((NUM_SEQS, N_HEADS, HEAD_SIZE),
                                       jnp.float32),
    )(context_lens, q3, tok, kact, vact)


# speedup vs baseline: 5.6962x; 1.1778x over previous
import math

import jax
import jax.numpy as jnp
from jax.experimental import pallas as pl
import jax.experimental.pallas.tpu as pltpu

N_HEADS = 16
N_KV_HEADS = 4
HEAD_SIZE = 128
MAX_SEQLEN = 2048
SPARSE_BLOCK = 64
VLLM_BLOCK = 16
LOCAL_BLOCKS = 8
VERT_STRIDE = 4
NUM_SEQS = 32
PAGES_PER_SPARSE = SPARSE_BLOCK // VLLM_BLOCK
NUM_SPARSE_BLOCKS = MAX_SEQLEN // SPARSE_BLOCK
MAX_ACTIVE = LOCAL_BLOCKS + (NUM_SPARSE_BLOCKS - LOCAL_BLOCKS) // VERT_STRIDE
PAGES_PER_SEQ = MAX_ACTIVE * PAGES_PER_SPARSE
T_ACT = MAX_ACTIVE * SPARSE_BLOCK

R = N_HEADS // N_KV_HEADS
SM_SCALE = 1.0 / math.sqrt(HEAD_SIZE)
NEG_INF = -1e30


def _attn_kernel(ctx_ref, q_ref, tok_ref, k_hbm, v_hbm, out_ref,
                 kbuf, vbuf, ksem, vsem):
    s = pl.program_id(0)
    slot = jax.lax.rem(s, 2)

    def _start(seq, slot):
        base = seq * PAGES_PER_SEQ
        pltpu.make_async_copy(k_hbm.at[pl.ds(base, PAGES_PER_SEQ)],
                              kbuf.at[slot], ksem.at[slot]).start()
        pltpu.make_async_copy(v_hbm.at[pl.ds(base, PAGES_PER_SEQ)],
                              vbuf.at[slot], vsem.at[slot]).start()

    @pl.when(s == 0)
    def _prologue():
        _start(s, slot)

    @pl.when(s + 1 < NUM_SEQS)
    def _prefetch_next():
        _start(s + 1, 1 - slot)

    pltpu.make_async_copy(k_hbm.at[pl.ds(0, PAGES_PER_SEQ)],
                          kbuf.at[slot], ksem.at[slot]).wait()
    pltpu.make_async_copy(v_hbm.at[pl.ds(0, PAGES_PER_SEQ)],
                          vbuf.at[slot], vsem.at[slot]).wait()

    valid = tok_ref[0, 0] < ctx_ref[s]
    outs = []
    for h in range(N_KV_HEADS):
        kh = kbuf[slot, :, h].reshape(T_ACT, HEAD_SIZE)
        vh = vbuf[slot, :, h].reshape(T_ACT, HEAD_SIZE)
        qh = q_ref[0, h]
        sc = jax.lax.dot_general(
            qh, kh,
            dimension_numbers=(((1,), (1,)), ((), ())),
            preferred_element_type=jnp.float32)
        sc = jnp.where(valid[None, :], sc, NEG_INF)
        m = jnp.max(sc, axis=1, keepdims=True)
        p = jnp.exp(sc - m)
        l = jnp.sum(p, axis=1, keepdims=True)
        pv = jax.lax.dot_general(
            p, vh,
            dimension_numbers=(((1,), (0,)), ((), ())),
            preferred_element_type=jnp.float32)
        outs.append(pv / l)
    out_ref[0] = jnp.concatenate(outs, axis=0)


@jax.jit
def kernel(q, k, v, block_tables, context_lens):
    qb = (context_lens - 1) // SPARSE_BLOCK
    n_str = jnp.maximum((qb - (LOCAL_BLOCKS - 1)) // VERT_STRIDE, 0)
    local_start = jnp.maximum(qb - (LOCAL_BLOCKS - 1), 0)
    n_act = n_str + jnp.minimum(qb + 1, LOCAL_BLOCKS)
    jj = jnp.arange(MAX_ACTIVE, dtype=jnp.int32)[None, :]
    blk = jnp.where(jj < n_str[:, None],
                    VERT_STRIDE * jj + (VERT_STRIDE - 1),
                    local_start[:, None] + (jj - n_str[:, None]))
    blk = jnp.minimum(blk, qb[:, None]).astype(jnp.int32)

    tok = (blk[:, :, None] * SPARSE_BLOCK
           + jnp.arange(SPARSE_BLOCK, dtype=jnp.int32))
    tok = jnp.where((jj < n_act[:, None])[:, :, None], tok, jnp.int32(1 << 30))
    tok = tok.reshape(NUM_SEQS, 1, T_ACT)

    first_page = jnp.take_along_axis(block_tables, blk * PAGES_PER_SPARSE,
                                     axis=1)
    first_page = jnp.where(jj < n_act[:, None], first_page,
                           block_tables[:, :1])
    pages = (first_page[..., None]
             + jnp.arange(PAGES_PER_SPARSE, dtype=jnp.int32))
    pages = pages.reshape(-1)
    kact = jnp.take(k, pages, axis=0).transpose(0, 1, 3, 2)
    vact = jnp.take(v, pages, axis=0).transpose(0, 1, 3, 2)

    q3 = (q * SM_SCALE).reshape(NUM_SEQS, N_KV_HEADS, R, HEAD_SIZE)

    grid_spec = pltpu.PrefetchScalarGridSpec(
        num_scalar_prefetch=1,
        grid=(NUM_SEQS,),
        in_specs=[
            pl.BlockSpec((1, N_KV_HEADS, R, HEAD_SIZE),
                         lambda s, *_: (s, 0, 0, 0)),
            pl.BlockSpec((1, 1, T_ACT), lambda s, *_: (s, 0, 0)),
            pl.BlockSpec(memory_space=pl.ANY),
            pl.BlockSpec(memory_space=pl.ANY),
        ],
        out_specs=pl.BlockSpec((1, N_HEADS, HEAD_SIZE),
                               lambda s, *_: (s, 0, 0)),
        scratch_shapes=[
            pltpu.VMEM((2, PAGES_PER_SEQ, N_KV_HEADS, VLLM_BLOCK,
                        HEAD_SIZE), jnp.float32),
            pltpu.VMEM((2, PAGES_PER_SEQ, N_KV_HEADS, VLLM_BLOCK,
                        HEAD_SIZE), jnp.float32),
            pltpu.SemaphoreType.DMA((2,)),
            pltpu.SemaphoreType.DMA((2,)),
        ],
    )

    return pl.pallas_call(
        _attn_kernel,
        grid_spec=grid_spec,
        out_shape=jax.ShapeDtypeStruct((NUM_SEQS, N_HEADS, HEAD_SIZE),
                                       jnp.float32),
    )(context_lens, q3, tok, kact, vact)
